# baseline (speedup 1.0000x reference)
import functools

import jax
import jax.numpy as jnp
from jax import lax
from jax.experimental import pallas as pl
from jax.experimental.pallas import tpu as pltpu
from jax.experimental.pallas import tpu_sc as plsc

_N = 10000
_E = 320000
_NPAD = 10112
_ROWS_PER_TILE = _NPAD // 16
_NW = 32
_G = 32
_K = 318
_ETOT = _NW * _K * _G

_MESH = plsc.VectorSubcoreMesh(core_axis_name="c", subcore_axis_name="s")


def _deg_kernel(dst_hbm, ones_hbm, zeros_hbm, out_hbm, idx_v, val_v, acc_sh):
    c = lax.axis_index("c")
    s = lax.axis_index("s")
    wid = c * 16 + s
    base = s * _ROWS_PER_TILE
    pltpu.sync_copy(zeros_hbm.at[pl.ds(base, _ROWS_PER_TILE)],
                    acc_sh.at[pl.ds(base, _ROWS_PER_TILE)])
    pltpu.sync_copy(ones_hbm, val_v)
    pltpu.sync_copy(dst_hbm.at[wid], idx_v)
    plsc.subcore_barrier()

    def body(j, carry):
        pltpu.sync_copy(val_v, acc_sh.at[idx_v.at[j]], add=True)
        return carry

    lax.fori_loop(0, _K, body, 0)
    plsc.subcore_barrier()
    pltpu.sync_copy(acc_sh.at[pl.ds(base, _ROWS_PER_TILE)],
                    out_hbm.at[pl.ds(c * _NPAD + base, _ROWS_PER_TILE)])


_deg_call = pl.kernel(
    _deg_kernel,
    out_type=jax.ShapeDtypeStruct((2 * _NPAD, 16), jnp.float32),
    mesh=_MESH,
    scratch_types=[
        pltpu.VMEM((_K, _G), jnp.int32),
        pltpu.VMEM((_G, 16), jnp.float32),
        pltpu.VMEM_SHARED((_NPAD, 16), jnp.float32),
    ],
    compiler_params=pltpu.CompilerParams(use_tc_tiling_on_sc=False),
)


_NBUF = 6
_LAG = 3
_T = _K // _NBUF


def _make_agg(D):
    def agg_kernel(table_hbm, src_hbm, dst_hbm, zeros_hbm, out_hbm,
                   sidx, didx, bufs, acc_sh, gsems, ssems):
        c = lax.axis_index("c")
        s = lax.axis_index("s")
        wid = c * 16 + s
        base = s * _ROWS_PER_TILE
        pltpu.sync_copy(zeros_hbm.at[pl.ds(base, _ROWS_PER_TILE)],
                        acc_sh.at[pl.ds(base, _ROWS_PER_TILE)])
        pltpu.sync_copy(src_hbm.at[wid], sidx)
        pltpu.sync_copy(dst_hbm.at[wid], didx)
        plsc.subcore_barrier()

        def gather(j, i):
            pltpu.async_copy(table_hbm.at[sidx.at[j]], bufs.at[i], gsems.at[i])

        def gather_wait(j, i):
            pltpu.make_async_copy(
                table_hbm.at[sidx.at[j]], bufs.at[i], gsems.at[i]).wait()

        def scatter(j, i):
            pltpu.async_copy(
                bufs.at[i], acc_sh.at[didx.at[j]], ssems.at[i], add=True)

        def scatter_wait(j, i):
            pltpu.make_async_copy(
                bufs.at[i], acc_sh.at[didx.at[j]], ssems.at[i]).wait()

        for i in range(_NBUF):
            gather(i, i)
            if i >= _LAG:
                gather_wait(i - _LAG, i - _LAG)
                scatter(i - _LAG, i - _LAG)

        def body(t, carry):
            for i in range(_NBUF):
                j = t * _NBUF + i
                scatter_wait(j - _NBUF, i)
                gather(j, i)
                jl = j - _LAG
                il = (i - _LAG) % _NBUF
                gather_wait(jl, il)
                scatter(jl, il)
            return carry

        lax.fori_loop(1, _T, body, 0)
        for j in range(_K - _LAG, _K):
            gather_wait(j, j % _NBUF)
            scatter(j, j % _NBUF)
        for j in range(_K - _NBUF, _K):
            scatter_wait(j, j % _NBUF)
        plsc.subcore_barrier()
        pltpu.sync_copy(acc_sh.at[pl.ds(base, _ROWS_PER_TILE)],
                        out_hbm.at[pl.ds(c * _NPAD + base, _ROWS_PER_TILE)])

    return pl.kernel(
        agg_kernel,
        out_type=jax.ShapeDtypeStruct((2 * _NPAD, D), jnp.float32),
        mesh=_MESH,
        scratch_types=[
            pltpu.VMEM((_K, _G), jnp.int32),
            pltpu.VMEM((_K, _G), jnp.int32),
            pltpu.VMEM((_NBUF, _G, D), jnp.float32),
            pltpu.VMEM_SHARED((_NPAD, D), jnp.float32),
            pltpu.SemaphoreType.DMA((_NBUF,)),
            pltpu.SemaphoreType.DMA((_NBUF,)),
        ],
        compiler_params=pltpu.CompilerParams(use_tc_tiling_on_sc=False),
    )


_agg128 = _make_agg(128)
_agg64 = _make_agg(64)


def _dinv(deg_ref):
    degsum = deg_ref[0:_N, 0:1] + deg_ref[_NPAD:_NPAD + _N, 0:1] + 1.0
    return lax.rsqrt(degsum)


def _tc_scale_xw(deg_ref, x_ref, w1_ref, hs_ref):
    xw = jnp.dot(x_ref[...], w1_ref[...], preferred_element_type=jnp.float32)
    hs_ref[...] = xw * _dinv(deg_ref)


def _tc_mid(deg_ref, p_ref, hs_ref, b1_ref, w2_ref, out_ref):
    dinv = _dinv(deg_ref)
    agg = p_ref[0:_N, :] + p_ref[_NPAD:_NPAD + _N, :] + hs_ref[...]
    h = jnp.maximum(agg * dinv + b1_ref[...], 0.0)
    out_ref[...] = jnp.dot(h, w2_ref[...],
                           preferred_element_type=jnp.float32) * dinv


def _tc_final(deg_ref, p_ref, h2s_ref, b2_ref, out_ref):
    dinv = _dinv(deg_ref)
    z = (p_ref[0:_N, :] + p_ref[_NPAD:_NPAD + _N, :] + h2s_ref[...]) * dinv \
        + b2_ref[...]
    m = jnp.max(z, axis=1, keepdims=True)
    lse = jnp.log(jnp.sum(jnp.exp(z - m), axis=1, keepdims=True)) + m
    out_ref[...] = z - lse


def kernel(x, edge_index, W1, b1, W2, b2):
    src = edge_index[0].astype(jnp.int32)
    dst = edge_index[1].astype(jnp.int32)
    pad = _ETOT - _E
    src_p = jnp.concatenate(
        [src, jnp.zeros((pad,), jnp.int32)]).reshape(_NW, _K, _G)
    dst_p = jnp.concatenate(
        [dst, jnp.full((pad,), _N, jnp.int32)]).reshape(_NW, _K, _G)
    ones16 = jnp.ones((_G, 16), jnp.float32)
    z16 = jnp.zeros((_NPAD, 16), jnp.float32)
    z128 = jnp.zeros((_NPAD, 128), jnp.float32)
    z64 = jnp.zeros((_NPAD, 64), jnp.float32)

    degp = _deg_call(dst_p, ones16, z16)

    hs = pl.pallas_call(
        _tc_scale_xw,
        out_shape=jax.ShapeDtypeStruct((_N, 128), jnp.float32),
    )(degp, x, W1)

    p1 = _agg128(hs, src_p, dst_p, z128)

    h2s = pl.pallas_call(
        _tc_mid,
        out_shape=jax.ShapeDtypeStruct((_N, 64), jnp.float32),
    )(degp, p1, hs, b1.reshape(1, 128), W2)

    p2 = _agg64(h2s, src_p, dst_p, z64)

    out = pl.pallas_call(
        _tc_final,
        out_shape=jax.ShapeDtypeStruct((_N, 64), jnp.float32),
    )(degp, p2, h2s, b2.reshape(1, 64))
    return out


# speedup vs baseline: 1.8902x; 1.8902x over previous
import functools

import jax
import jax.numpy as jnp
from jax import lax
from jax.experimental import pallas as pl
from jax.experimental.pallas import tpu as pltpu
from jax.experimental.pallas import tpu_sc as plsc

_N = 10000
_E = 320000
_NPAD = 10112
_ROWS_PER_TILE = _NPAD // 16
_NW = 32
_G = 56
_K = 180
_ETOT = _NW * _K * _G

_MESH = plsc.VectorSubcoreMesh(core_axis_name="c", subcore_axis_name="s")


def _deg_kernel(dst_hbm, ones_hbm, zeros_hbm, out_hbm, idx_v, val_v, acc_sh):
    c = lax.axis_index("c")
    s = lax.axis_index("s")
    wid = c * 16 + s
    base = s * _ROWS_PER_TILE
    pltpu.sync_copy(zeros_hbm.at[pl.ds(base, _ROWS_PER_TILE)],
                    acc_sh.at[pl.ds(base, _ROWS_PER_TILE)])
    pltpu.sync_copy(ones_hbm, val_v)
    pltpu.sync_copy(dst_hbm.at[wid], idx_v)
    plsc.subcore_barrier()

    def body(j, carry):
        pltpu.sync_copy(val_v, acc_sh.at[idx_v.at[j]], add=True)
        return carry

    lax.fori_loop(0, _K, body, 0)
    plsc.subcore_barrier()
    pltpu.sync_copy(acc_sh.at[pl.ds(base, _ROWS_PER_TILE)],
                    out_hbm.at[pl.ds(c * _NPAD + base, _ROWS_PER_TILE)])


_deg_call = pl.kernel(
    _deg_kernel,
    out_type=jax.ShapeDtypeStruct((2 * _NPAD, 16), jnp.float32),
    mesh=_MESH,
    scratch_types=[
        pltpu.VMEM((_K, _G), jnp.int32),
        pltpu.VMEM((_G, 16), jnp.float32),
        pltpu.VMEM_SHARED((_NPAD, 16), jnp.float32),
    ],
    compiler_params=pltpu.CompilerParams(use_tc_tiling_on_sc=False),
)


_NBUF = 4
_LAG = 3
_T = _K // _NBUF


def _make_agg(D):
    def agg_kernel(table_hbm, src_hbm, dst_hbm, zeros_hbm, out_hbm,
                   sidx, didx, bufs, acc_sh, gsems, ssems):
        c = lax.axis_index("c")
        s = lax.axis_index("s")
        wid = c * 16 + s
        base = s * _ROWS_PER_TILE
        pltpu.sync_copy(zeros_hbm.at[pl.ds(base, _ROWS_PER_TILE)],
                        acc_sh.at[pl.ds(base, _ROWS_PER_TILE)])
        pltpu.sync_copy(src_hbm.at[wid], sidx)
        pltpu.sync_copy(dst_hbm.at[wid], didx)
        plsc.subcore_barrier()

        def gather(j, i):
            pltpu.async_copy(table_hbm.at[sidx.at[j]], bufs.at[i], gsems.at[i])

        def gather_wait(j, i):
            pltpu.make_async_copy(
                table_hbm.at[sidx.at[j]], bufs.at[i], gsems.at[i]).wait()

        def scatter(j, i):
            pltpu.async_copy(
                bufs.at[i], acc_sh.at[didx.at[j]], ssems.at[i], add=True)

        def scatter_wait(j, i):
            pltpu.make_async_copy(
                bufs.at[i], acc_sh.at[didx.at[j]], ssems.at[i]).wait()

        for i in range(_NBUF):
            gather(i, i)
            if i >= _LAG:
                gather_wait(i - _LAG, i - _LAG)
                scatter(i - _LAG, i - _LAG)

        def body(t, carry):
            for i in range(_NBUF):
                j = t * _NBUF + i
                scatter_wait(j - _NBUF, i)
                gather(j, i)
                jl = j - _LAG
                il = (i - _LAG) % _NBUF
                gather_wait(jl, il)
                scatter(jl, il)
            return carry

        lax.fori_loop(1, _T, body, 0)
        for j in range(_K - _LAG, _K):
            gather_wait(j, j % _NBUF)
            scatter(j, j % _NBUF)
        for j in range(_K - _NBUF, _K):
            scatter_wait(j, j % _NBUF)
        plsc.subcore_barrier()
        pltpu.sync_copy(acc_sh.at[pl.ds(base, _ROWS_PER_TILE)],
                        out_hbm.at[pl.ds(c * _NPAD + base, _ROWS_PER_TILE)])

    return pl.kernel(
        agg_kernel,
        out_type=jax.ShapeDtypeStruct((2 * _NPAD, D), jnp.float32),
        mesh=_MESH,
        scratch_types=[
            pltpu.VMEM((_K, _G), jnp.int32),
            pltpu.VMEM((_K, _G), jnp.int32),
            pltpu.VMEM((_NBUF, _G, D), jnp.float32),
            pltpu.VMEM_SHARED((_NPAD, D), jnp.float32),
            pltpu.SemaphoreType.DMA((_NBUF,)),
            pltpu.SemaphoreType.DMA((_NBUF,)),
        ],
        compiler_params=pltpu.CompilerParams(use_tc_tiling_on_sc=False),
    )


_agg128 = _make_agg(128)
_agg64 = _make_agg(64)


def _make_agg_sp(D):
    def agg_kernel(table_hbm, src_hbm, dst_hbm, zeros_hbm, out_hbm,
                   sidx, didx, bufs, table_sh, acc_sh, gsems, ssems):
        c = lax.axis_index("c")
        s = lax.axis_index("s")
        wid = c * 16 + s
        base = s * _ROWS_PER_TILE
        pltpu.sync_copy(zeros_hbm.at[pl.ds(base, _ROWS_PER_TILE)],
                        acc_sh.at[pl.ds(base, _ROWS_PER_TILE)])
        pltpu.sync_copy(table_hbm.at[pl.ds(base, _ROWS_PER_TILE)],
                        table_sh.at[pl.ds(base, _ROWS_PER_TILE)])
        pltpu.sync_copy(src_hbm.at[wid], sidx)
        pltpu.sync_copy(dst_hbm.at[wid], didx)
        plsc.subcore_barrier()

        def gather(j, i):
            pltpu.async_copy(table_sh.at[sidx.at[j]], bufs.at[i], gsems.at[i])

        def gather_wait(j, i):
            pltpu.make_async_copy(
                table_sh.at[sidx.at[j]], bufs.at[i], gsems.at[i]).wait()

        def scatter(j, i):
            pltpu.async_copy(
                bufs.at[i], acc_sh.at[didx.at[j]], ssems.at[i], add=True)

        def scatter_wait(j, i):
            pltpu.make_async_copy(
                bufs.at[i], acc_sh.at[didx.at[j]], ssems.at[i]).wait()

        for i in range(_NBUF):
            gather(i, i)
            if i >= _LAG:
                gather_wait(i - _LAG, i - _LAG)
                scatter(i - _LAG, i - _LAG)

        def body(t, carry):
            for i in range(_NBUF):
                j = t * _NBUF + i
                scatter_wait(j - _NBUF, i)
                gather(j, i)
                jl = j - _LAG
                il = (i - _LAG) % _NBUF
                gather_wait(jl, il)
                scatter(jl, il)
            return carry

        lax.fori_loop(1, _T, body, 0)
        for j in range(_K - _LAG, _K):
            gather_wait(j, j % _NBUF)
            scatter(j, j % _NBUF)
        for j in range(_K - _NBUF, _K):
            scatter_wait(j, j % _NBUF)
        plsc.subcore_barrier()
        pltpu.sync_copy(acc_sh.at[pl.ds(base, _ROWS_PER_TILE)],
                        out_hbm.at[pl.ds(c * _NPAD + base, _ROWS_PER_TILE)])

    return pl.kernel(
        agg_kernel,
        out_type=jax.ShapeDtypeStruct((2 * _NPAD, D), jnp.float32),
        mesh=_MESH,
        scratch_types=[
            pltpu.VMEM((_K, _G), jnp.int32),
            pltpu.VMEM((_K, _G), jnp.int32),
            pltpu.VMEM((_NBUF, _G, D), jnp.float32),
            pltpu.VMEM_SHARED((_NPAD, D), jnp.float32),
            pltpu.VMEM_SHARED((_NPAD, D), jnp.float32),
            pltpu.SemaphoreType.DMA((_NBUF,)),
            pltpu.SemaphoreType.DMA((_NBUF,)),
        ],
        compiler_params=pltpu.CompilerParams(use_tc_tiling_on_sc=False),
    )


_agg64_sp = _make_agg_sp(64)


def _dinv(deg_ref):
    degsum = deg_ref[0:_N, 0:1] + deg_ref[_NPAD:_NPAD + _N, 0:1] + 1.0
    return lax.rsqrt(degsum)


def _tc_scale_xw(deg_ref, x_ref, w1_ref, hs_ref):
    xw = jnp.dot(x_ref[...], w1_ref[...], preferred_element_type=jnp.float32)
    hs_ref[...] = xw * _dinv(deg_ref)


def _tc_mid(deg_ref, p_ref, hs_ref, b1_ref, w2_ref, out_ref):
    dinv = _dinv(deg_ref)
    agg = p_ref[0:_N, :] + p_ref[_NPAD:_NPAD + _N, :] + hs_ref[...]
    h = jnp.maximum(agg * dinv + b1_ref[...], 0.0)
    out_ref[0:_N, :] = jnp.dot(h, w2_ref[...],
                               preferred_element_type=jnp.float32) * dinv
    out_ref[_N:_NPAD, :] = jnp.zeros((_NPAD - _N, 64), jnp.float32)


def _tc_final(deg_ref, p_ref, h2s_ref, b2_ref, out_ref):
    dinv = _dinv(deg_ref)
    z = (p_ref[0:_N, :] + p_ref[_NPAD:_NPAD + _N, :] + h2s_ref[0:_N, :]) \
        * dinv + b2_ref[...]
    m = jnp.max(z, axis=1, keepdims=True)
    lse = jnp.log(jnp.sum(jnp.exp(z - m), axis=1, keepdims=True)) + m
    out_ref[...] = z - lse


def kernel(x, edge_index, W1, b1, W2, b2):
    src = edge_index[0].astype(jnp.int32)
    dst = edge_index[1].astype(jnp.int32)
    pad = _ETOT - _E
    src_p = jnp.concatenate(
        [src, jnp.zeros((pad,), jnp.int32)]).reshape(_NW, _K, _G)
    dst_p = jnp.concatenate(
        [dst, jnp.full((pad,), _N, jnp.int32)]).reshape(_NW, _K, _G)
    ones16 = jnp.ones((_G, 16), jnp.float32)
    z16 = jnp.zeros((_NPAD, 16), jnp.float32)
    z128 = jnp.zeros((_NPAD, 128), jnp.float32)
    z64 = jnp.zeros((_NPAD, 64), jnp.float32)

    degp = _deg_call(dst_p, ones16, z16)

    hs = pl.pallas_call(
        _tc_scale_xw,
        out_shape=jax.ShapeDtypeStruct((_N, 128), jnp.float32),
    )(degp, x, W1)

    p1 = _agg128(hs, src_p, dst_p, z128)

    h2s = pl.pallas_call(
        _tc_mid,
        out_shape=jax.ShapeDtypeStruct((_NPAD, 64), jnp.float32),
    )(degp, p1, hs, b1.reshape(1, 128), W2)

    p2 = _agg64_sp(h2s, src_p, dst_p, z64)

    out = pl.pallas_call(
        _tc_final,
        out_shape=jax.ShapeDtypeStruct((_N, 64), jnp.float32),
    )(degp, p2, h2s, b2.reshape(1, 64))
    return out


# speedup vs baseline: 2.0972x; 1.1095x over previous
import functools

import jax
import jax.numpy as jnp
from jax import lax
from jax.experimental import pallas as pl
from jax.experimental.pallas import tpu as pltpu
from jax.experimental.pallas import tpu_sc as plsc

_N = 10000
_E = 320000
_NPAD = 10112
_ROWS_PER_TILE = _NPAD // 16
_NW = 32
_G = 56
_K = 180
_ETOT = _NW * _K * _G

_MESH = plsc.VectorSubcoreMesh(core_axis_name="c", subcore_axis_name="s")


def _deg_kernel(dst_hbm, ones_hbm, zeros_hbm, out_hbm, idx_v, val_v, acc_sh):
    c = lax.axis_index("c")
    s = lax.axis_index("s")
    wid = c * 16 + s
    base = s * _ROWS_PER_TILE
    pltpu.sync_copy(zeros_hbm.at[pl.ds(base, _ROWS_PER_TILE)],
                    acc_sh.at[pl.ds(base, _ROWS_PER_TILE)])
    pltpu.sync_copy(ones_hbm, val_v)
    pltpu.sync_copy(dst_hbm.at[wid], idx_v)
    plsc.subcore_barrier()

    def body(j, carry):
        pltpu.sync_copy(val_v, acc_sh.at[idx_v.at[j]], add=True)
        return carry

    lax.fori_loop(0, _K, body, 0)
    plsc.subcore_barrier()
    pltpu.sync_copy(acc_sh.at[pl.ds(base, _ROWS_PER_TILE)],
                    out_hbm.at[pl.ds(c * _NPAD + base, _ROWS_PER_TILE)])


_deg_call = pl.kernel(
    _deg_kernel,
    out_type=jax.ShapeDtypeStruct((2 * _NPAD, 16), jnp.float32),
    mesh=_MESH,
    scratch_types=[
        pltpu.VMEM((_K, _G), jnp.int32),
        pltpu.VMEM((_G, 16), jnp.float32),
        pltpu.VMEM_SHARED((_NPAD, 16), jnp.float32),
    ],
    compiler_params=pltpu.CompilerParams(use_tc_tiling_on_sc=False),
)


_NBUF = 4
_LAG = 3
_T = _K // _NBUF


def _make_agg(D):
    def agg_kernel(table_hbm, src_hbm, dst_hbm, zeros_hbm, out_hbm,
                   sidx, didx, bufs, acc_sh, gsems, ssems):
        c = lax.axis_index("c")
        s = lax.axis_index("s")
        wid = c * 16 + s
        base = s * _ROWS_PER_TILE
        pltpu.sync_copy(zeros_hbm.at[pl.ds(base, _ROWS_PER_TILE)],
                        acc_sh.at[pl.ds(base, _ROWS_PER_TILE)])
        pltpu.sync_copy(src_hbm.at[wid], sidx)
        pltpu.sync_copy(dst_hbm.at[wid], didx)
        plsc.subcore_barrier()

        def gather(j, i):
            pltpu.async_copy(table_hbm.at[sidx.at[j]], bufs.at[i], gsems.at[i])

        def gather_wait(j, i):
            pltpu.make_async_copy(
                table_hbm.at[sidx.at[j]], bufs.at[i], gsems.at[i]).wait()

        def scatter(j, i):
            pltpu.async_copy(
                bufs.at[i], acc_sh.at[didx.at[j]], ssems.at[i], add=True)

        def scatter_wait(j, i):
            pltpu.make_async_copy(
                bufs.at[i], acc_sh.at[didx.at[j]], ssems.at[i]).wait()

        for i in range(_NBUF):
            gather(i, i)
            if i >= _LAG:
                gather_wait(i - _LAG, i - _LAG)
                scatter(i - _LAG, i - _LAG)

        def body(t, carry):
            for i in range(_NBUF):
                j = t * _NBUF + i
                scatter_wait(j - _NBUF, i)
                gather(j, i)
                jl = j - _LAG
                il = (i - _LAG) % _NBUF
                gather_wait(jl, il)
                scatter(jl, il)
            return carry

        lax.fori_loop(1, _T, body, 0)
        for j in range(_K - _LAG, _K):
            gather_wait(j, j % _NBUF)
            scatter(j, j % _NBUF)
        for j in range(_K - _NBUF, _K):
            scatter_wait(j, j % _NBUF)
        plsc.subcore_barrier()
        pltpu.sync_copy(acc_sh.at[pl.ds(base, _ROWS_PER_TILE)],
                        out_hbm.at[pl.ds(c * _NPAD + base, _ROWS_PER_TILE)])

    return pl.kernel(
        agg_kernel,
        out_type=jax.ShapeDtypeStruct((2 * _NPAD, D), jnp.float32),
        mesh=_MESH,
        scratch_types=[
            pltpu.VMEM((_K, _G), jnp.int32),
            pltpu.VMEM((_K, _G), jnp.int32),
            pltpu.VMEM((_NBUF, _G, D), jnp.float32),
            pltpu.VMEM_SHARED((_NPAD, D), jnp.float32),
            pltpu.SemaphoreType.DMA((_NBUF,)),
            pltpu.SemaphoreType.DMA((_NBUF,)),
        ],
        compiler_params=pltpu.CompilerParams(use_tc_tiling_on_sc=False),
    )


_agg128 = _make_agg(128)
_agg64 = _make_agg(64)


def _make_agg_sp(D):
    def agg_kernel(table_hbm, src_hbm, dst_hbm, zeros_hbm, out_hbm,
                   sidx, didx, bufs, table_sh, acc_sh, gsems, ssems):
        c = lax.axis_index("c")
        s = lax.axis_index("s")
        wid = c * 16 + s
        base = s * _ROWS_PER_TILE
        pltpu.sync_copy(zeros_hbm.at[pl.ds(base, _ROWS_PER_TILE)],
                        acc_sh.at[pl.ds(base, _ROWS_PER_TILE)])
        pltpu.sync_copy(table_hbm.at[pl.ds(base, _ROWS_PER_TILE)],
                        table_sh.at[pl.ds(base, _ROWS_PER_TILE)])
        pltpu.sync_copy(src_hbm.at[wid], sidx)
        pltpu.sync_copy(dst_hbm.at[wid], didx)
        plsc.subcore_barrier()

        def gather(j, i):
            pltpu.async_copy(table_sh.at[sidx.at[j]], bufs.at[i], gsems.at[i])

        def gather_wait(j, i):
            pltpu.make_async_copy(
                table_sh.at[sidx.at[j]], bufs.at[i], gsems.at[i]).wait()

        def scatter(j, i):
            pltpu.async_copy(
                bufs.at[i], acc_sh.at[didx.at[j]], ssems.at[i], add=True)

        def scatter_wait(j, i):
            pltpu.make_async_copy(
                bufs.at[i], acc_sh.at[didx.at[j]], ssems.at[i]).wait()

        for i in range(_NBUF):
            gather(i, i)
            if i >= _LAG:
                gather_wait(i - _LAG, i - _LAG)
                scatter(i - _LAG, i - _LAG)

        def body(t, carry):
            for i in range(_NBUF):
                j = t * _NBUF + i
                scatter_wait(j - _NBUF, i)
                gather(j, i)
                jl = j - _LAG
                il = (i - _LAG) % _NBUF
                gather_wait(jl, il)
                scatter(jl, il)
            return carry

        lax.fori_loop(1, _T, body, 0)
        for j in range(_K - _LAG, _K):
            gather_wait(j, j % _NBUF)
            scatter(j, j % _NBUF)
        for j in range(_K - _NBUF, _K):
            scatter_wait(j, j % _NBUF)
        plsc.subcore_barrier()
        pltpu.sync_copy(acc_sh.at[pl.ds(base, _ROWS_PER_TILE)],
                        out_hbm.at[pl.ds(c * _NPAD + base, _ROWS_PER_TILE)])

    return pl.kernel(
        agg_kernel,
        out_type=jax.ShapeDtypeStruct((2 * _NPAD, D), jnp.float32),
        mesh=_MESH,
        scratch_types=[
            pltpu.VMEM((_K, _G), jnp.int32),
            pltpu.VMEM((_K, _G), jnp.int32),
            pltpu.VMEM((_NBUF, _G, D), jnp.float32),
            pltpu.VMEM_SHARED((_NPAD, D), jnp.float32),
            pltpu.VMEM_SHARED((_NPAD, D), jnp.float32),
            pltpu.SemaphoreType.DMA((_NBUF,)),
            pltpu.SemaphoreType.DMA((_NBUF,)),
        ],
        compiler_params=pltpu.CompilerParams(use_tc_tiling_on_sc=False),
    )


_agg64_sp = _make_agg_sp(64)


def _dinv(deg_ref):
    degsum = deg_ref[0:_N, 0:1] + deg_ref[_NPAD:_NPAD + _N, 0:1] + 1.0
    return lax.rsqrt(degsum)


def _tc_scale_xw(deg_ref, x_ref, w1_ref, hsa_ref, hsb_ref):
    xw = jnp.dot(x_ref[...], w1_ref[...], preferred_element_type=jnp.float32)
    hs = xw * _dinv(deg_ref)
    zpad = jnp.zeros((_NPAD - _N, 64), jnp.float32)
    hsa_ref[0:_N, :] = hs[:, 0:64]
    hsa_ref[_N:_NPAD, :] = zpad
    hsb_ref[0:_N, :] = hs[:, 64:128]
    hsb_ref[_N:_NPAD, :] = zpad


def _tc_mid(deg_ref, pa_ref, pb_ref, hsa_ref, hsb_ref, b1_ref, w2_ref,
            out_ref):
    dinv = _dinv(deg_ref)
    agg_a = pa_ref[0:_N, :] + pa_ref[_NPAD:_NPAD + _N, :] + hsa_ref[0:_N, :]
    agg_b = pb_ref[0:_N, :] + pb_ref[_NPAD:_NPAD + _N, :] + hsb_ref[0:_N, :]
    agg = jnp.concatenate([agg_a, agg_b], axis=1)
    h = jnp.maximum(agg * dinv + b1_ref[...], 0.0)
    out_ref[0:_N, :] = jnp.dot(h, w2_ref[...],
                               preferred_element_type=jnp.float32) * dinv
    out_ref[_N:_NPAD, :] = jnp.zeros((_NPAD - _N, 64), jnp.float32)


def _tc_final(deg_ref, p_ref, h2s_ref, b2_ref, out_ref):
    dinv = _dinv(deg_ref)
    z = (p_ref[0:_N, :] + p_ref[_NPAD:_NPAD + _N, :] + h2s_ref[0:_N, :]) \
        * dinv + b2_ref[...]
    m = jnp.max(z, axis=1, keepdims=True)
    lse = jnp.log(jnp.sum(jnp.exp(z - m), axis=1, keepdims=True)) + m
    out_ref[...] = z - lse


def kernel(x, edge_index, W1, b1, W2, b2):
    src = edge_index[0].astype(jnp.int32)
    dst = edge_index[1].astype(jnp.int32)
    pad = _ETOT - _E
    src_p = jnp.concatenate(
        [src, jnp.zeros((pad,), jnp.int32)]).reshape(_NW, _K, _G)
    dst_p = jnp.concatenate(
        [dst, jnp.full((pad,), _N, jnp.int32)]).reshape(_NW, _K, _G)
    ones16 = jnp.ones((_G, 16), jnp.float32)
    z16 = jnp.zeros((_NPAD, 16), jnp.float32)
    z64 = jnp.zeros((_NPAD, 64), jnp.float32)

    degp = _deg_call(dst_p, ones16, z16)

    hsa, hsb = pl.pallas_call(
        _tc_scale_xw,
        out_shape=(jax.ShapeDtypeStruct((_NPAD, 64), jnp.float32),
                   jax.ShapeDtypeStruct((_NPAD, 64), jnp.float32)),
    )(degp, x, W1)

    p1a = _agg64_sp(hsa, src_p, dst_p, z64)
    p1b = _agg64_sp(hsb, src_p, dst_p, z64)

    h2s = pl.pallas_call(
        _tc_mid,
        out_shape=jax.ShapeDtypeStruct((_NPAD, 64), jnp.float32),
    )(degp, p1a, p1b, hsa, hsb, b1.reshape(1, 128), W2)

    p2 = _agg64_sp(h2s, src_p, dst_p, z64)

    out = pl.pallas_call(
        _tc_final,
        out_shape=jax.ShapeDtypeStruct((_N, 64), jnp.float32),
    )(degp, p2, h2s, b2.reshape(1, 64))
    return out


# speedup vs baseline: 2.1544x; 1.0273x over previous
import functools

import jax
import jax.numpy as jnp
from jax import lax
from jax.experimental import pallas as pl
from jax.experimental.pallas import tpu as pltpu
from jax.experimental.pallas import tpu_sc as plsc

_N = 10000
_E = 320000
_NPAD = 10112
_ROWS_PER_TILE = _NPAD // 16
_NW = 32
_G = 56
_K = 180
_ETOT = _NW * _K * _G

_MESH = plsc.VectorSubcoreMesh(core_axis_name="c", subcore_axis_name="s")


def _deg_kernel(dst_hbm, ones_hbm, zeros_hbm, out_hbm, idx_v, val_v, acc_sh,
                sem):
    c = lax.axis_index("c")
    s = lax.axis_index("s")
    wid = c * 16 + s
    base = s * _ROWS_PER_TILE
    pltpu.sync_copy(zeros_hbm.at[pl.ds(base, _ROWS_PER_TILE)],
                    acc_sh.at[pl.ds(base, _ROWS_PER_TILE)])
    pltpu.sync_copy(ones_hbm, val_v)
    pltpu.sync_copy(dst_hbm.at[wid], idx_v)
    plsc.subcore_barrier()

    def fire(j, carry):
        pltpu.async_copy(val_v, acc_sh.at[idx_v.at[j]], sem, add=True)
        return carry

    def fire_drain(j, carry):
        pltpu.async_copy(val_v, acc_sh.at[idx_v.at[j]], sem, add=True)
        pltpu.make_async_copy(val_v, acc_sh.at[idx_v.at[j - 30]], sem).wait()
        return carry

    def drain(j, carry):
        pltpu.make_async_copy(val_v, acc_sh.at[idx_v.at[j]], sem).wait()
        return carry

    lax.fori_loop(0, 30, fire, 0)
    lax.fori_loop(30, _K, fire_drain, 0)
    lax.fori_loop(_K - 30, _K, drain, 0)
    plsc.subcore_barrier()
    pltpu.sync_copy(acc_sh.at[pl.ds(base, _ROWS_PER_TILE)],
                    out_hbm.at[pl.ds(c * _NPAD + base, _ROWS_PER_TILE)])


_deg_call = pl.kernel(
    _deg_kernel,
    out_type=jax.ShapeDtypeStruct((2 * _NPAD, 16), jnp.float32),
    mesh=_MESH,
    scratch_types=[
        pltpu.VMEM((_K, _G), jnp.int32),
        pltpu.VMEM((_G, 16), jnp.float32),
        pltpu.VMEM_SHARED((_NPAD, 16), jnp.float32),
        pltpu.SemaphoreType.DMA,
    ],
    compiler_params=pltpu.CompilerParams(use_tc_tiling_on_sc=False),
)


_NBUF = 4
_LAG = 3
_T = _K // _NBUF


def _make_agg(D):
    def agg_kernel(table_hbm, src_hbm, dst_hbm, zeros_hbm, out_hbm,
                   sidx, didx, bufs, acc_sh, gsems, ssems):
        c = lax.axis_index("c")
        s = lax.axis_index("s")
        wid = c * 16 + s
        base = s * _ROWS_PER_TILE
        pltpu.sync_copy(zeros_hbm.at[pl.ds(base, _ROWS_PER_TILE)],
                        acc_sh.at[pl.ds(base, _ROWS_PER_TILE)])
        pltpu.sync_copy(src_hbm.at[wid], sidx)
        pltpu.sync_copy(dst_hbm.at[wid], didx)
        plsc.subcore_barrier()

        def gather(j, i):
            pltpu.async_copy(table_hbm.at[sidx.at[j]], bufs.at[i], gsems.at[i])

        def gather_wait(j, i):
            pltpu.make_async_copy(
                table_hbm.at[sidx.at[j]], bufs.at[i], gsems.at[i]).wait()

        def scatter(j, i):
            pltpu.async_copy(
                bufs.at[i], acc_sh.at[didx.at[j]], ssems.at[i], add=True)

        def scatter_wait(j, i):
            pltpu.make_async_copy(
                bufs.at[i], acc_sh.at[didx.at[j]], ssems.at[i]).wait()

        for i in range(_NBUF):
            gather(i, i)
            if i >= _LAG:
                gather_wait(i - _LAG, i - _LAG)
                scatter(i - _LAG, i - _LAG)

        def body(t, carry):
            for i in range(_NBUF):
                j = t * _NBUF + i
                scatter_wait(j - _NBUF, i)
                gather(j, i)
                jl = j - _LAG
                il = (i - _LAG) % _NBUF
                gather_wait(jl, il)
                scatter(jl, il)
            return carry

        lax.fori_loop(1, _T, body, 0)
        for j in range(_K - _LAG, _K):
            gather_wait(j, j % _NBUF)
            scatter(j, j % _NBUF)
        for j in range(_K - _NBUF, _K):
            scatter_wait(j, j % _NBUF)
        plsc.subcore_barrier()
        pltpu.sync_copy(acc_sh.at[pl.ds(base, _ROWS_PER_TILE)],
                        out_hbm.at[pl.ds(c * _NPAD + base, _ROWS_PER_TILE)])

    return pl.kernel(
        agg_kernel,
        out_type=jax.ShapeDtypeStruct((2 * _NPAD, D), jnp.float32),
        mesh=_MESH,
        scratch_types=[
            pltpu.VMEM((_K, _G), jnp.int32),
            pltpu.VMEM((_K, _G), jnp.int32),
            pltpu.VMEM((_NBUF, _G, D), jnp.float32),
            pltpu.VMEM_SHARED((_NPAD, D), jnp.float32),
            pltpu.SemaphoreType.DMA((_NBUF,)),
            pltpu.SemaphoreType.DMA((_NBUF,)),
        ],
        compiler_params=pltpu.CompilerParams(use_tc_tiling_on_sc=False),
    )


_agg128 = _make_agg(128)
_agg64 = _make_agg(64)


def _make_agg_sp(D):
    def agg_kernel(table_hbm, src_hbm, dst_hbm, zeros_hbm, out_hbm,
                   sidx, didx, bufs, table_sh, acc_sh, gsems, ssems):
        c = lax.axis_index("c")
        s = lax.axis_index("s")
        wid = c * 16 + s
        base = s * _ROWS_PER_TILE
        pltpu.sync_copy(zeros_hbm.at[pl.ds(base, _ROWS_PER_TILE)],
                        acc_sh.at[pl.ds(base, _ROWS_PER_TILE)])
        pltpu.sync_copy(table_hbm.at[pl.ds(base, _ROWS_PER_TILE)],
                        table_sh.at[pl.ds(base, _ROWS_PER_TILE)])
        pltpu.sync_copy(src_hbm.at[wid], sidx)
        pltpu.sync_copy(dst_hbm.at[wid], didx)
        plsc.subcore_barrier()

        def gather(j, i):
            pltpu.async_copy(table_sh.at[sidx.at[j]], bufs.at[i], gsems.at[i])

        def gather_wait(j, i):
            pltpu.make_async_copy(
                table_sh.at[sidx.at[j]], bufs.at[i], gsems.at[i]).wait()

        def scatter(j, i):
            pltpu.async_copy(
                bufs.at[i], acc_sh.at[didx.at[j]], ssems.at[i], add=True)

        def scatter_wait(j, i):
            pltpu.make_async_copy(
                bufs.at[i], acc_sh.at[didx.at[j]], ssems.at[i]).wait()

        for i in range(_NBUF):
            gather(i, i)
            if i >= _LAG:
                gather_wait(i - _LAG, i - _LAG)
                scatter(i - _LAG, i - _LAG)

        def body(t, carry):
            for i in range(_NBUF):
                j = t * _NBUF + i
                scatter_wait(j - _NBUF, i)
                gather(j, i)
                jl = j - _LAG
                il = (i - _LAG) % _NBUF
                gather_wait(jl, il)
                scatter(jl, il)
            return carry

        lax.fori_loop(1, _T, body, 0)
        for j in range(_K - _LAG, _K):
            gather_wait(j, j % _NBUF)
            scatter(j, j % _NBUF)
        for j in range(_K - _NBUF, _K):
            scatter_wait(j, j % _NBUF)
        plsc.subcore_barrier()
        pltpu.sync_copy(acc_sh.at[pl.ds(base, _ROWS_PER_TILE)],
                        out_hbm.at[pl.ds(c * _NPAD + base, _ROWS_PER_TILE)])

    return pl.kernel(
        agg_kernel,
        out_type=jax.ShapeDtypeStruct((2 * _NPAD, D), jnp.float32),
        mesh=_MESH,
        scratch_types=[
            pltpu.VMEM((_K, _G), jnp.int32),
            pltpu.VMEM((_K, _G), jnp.int32),
            pltpu.VMEM((_NBUF, _G, D), jnp.float32),
            pltpu.VMEM_SHARED((_NPAD, D), jnp.float32),
            pltpu.VMEM_SHARED((_NPAD, D), jnp.float32),
            pltpu.SemaphoreType.DMA((_NBUF,)),
            pltpu.SemaphoreType.DMA((_NBUF,)),
        ],
        compiler_params=pltpu.CompilerParams(use_tc_tiling_on_sc=False),
    )


_agg64_sp = _make_agg_sp(64)


def _agg_sp2_kernel(ta_hbm, tb_hbm, src_hbm, dst_hbm, zeros_hbm,
                    outa_hbm, outb_hbm,
                    sidx, didx, bufs, table_sh, acc_sh, gsems, ssems):
    c = lax.axis_index("c")
    s = lax.axis_index("s")
    wid = c * 16 + s
    base = s * _ROWS_PER_TILE
    pltpu.sync_copy(zeros_hbm.at[pl.ds(base, _ROWS_PER_TILE)],
                    acc_sh.at[pl.ds(base, _ROWS_PER_TILE)])
    pltpu.sync_copy(ta_hbm.at[pl.ds(base, _ROWS_PER_TILE)],
                    table_sh.at[pl.ds(base, _ROWS_PER_TILE)])
    pltpu.sync_copy(src_hbm.at[wid], sidx)
    pltpu.sync_copy(dst_hbm.at[wid], didx)
    plsc.subcore_barrier()

    def gather(j, i):
        pltpu.async_copy(table_sh.at[sidx.at[j]], bufs.at[i], gsems.at[i])

    def gather_wait(j, i):
        pltpu.make_async_copy(
            table_sh.at[sidx.at[j]], bufs.at[i], gsems.at[i]).wait()

    def scatter(j, i):
        pltpu.async_copy(
            bufs.at[i], acc_sh.at[didx.at[j]], ssems.at[i], add=True)

    def scatter_wait(j, i):
        pltpu.make_async_copy(
            bufs.at[i], acc_sh.at[didx.at[j]], ssems.at[i]).wait()

    def pipeline():
        for i in range(_NBUF):
            gather(i, i)
            if i >= _LAG:
                gather_wait(i - _LAG, i - _LAG)
                scatter(i - _LAG, i - _LAG)

        def body(t, carry):
            for i in range(_NBUF):
                j = t * _NBUF + i
                scatter_wait(j - _NBUF, i)
                gather(j, i)
                jl = j - _LAG
                il = (i - _LAG) % _NBUF
                gather_wait(jl, il)
                scatter(jl, il)
            return carry

        lax.fori_loop(1, _T, body, 0)
        for j in range(_K - _LAG, _K):
            gather_wait(j, j % _NBUF)
            scatter(j, j % _NBUF)
        for j in range(_K - _NBUF, _K):
            scatter_wait(j, j % _NBUF)
        plsc.subcore_barrier()

    pipeline()
    pltpu.sync_copy(acc_sh.at[pl.ds(base, _ROWS_PER_TILE)],
                    outa_hbm.at[pl.ds(c * _NPAD + base, _ROWS_PER_TILE)])
    pltpu.sync_copy(zeros_hbm.at[pl.ds(base, _ROWS_PER_TILE)],
                    acc_sh.at[pl.ds(base, _ROWS_PER_TILE)])
    pltpu.sync_copy(tb_hbm.at[pl.ds(base, _ROWS_PER_TILE)],
                    table_sh.at[pl.ds(base, _ROWS_PER_TILE)])
    plsc.subcore_barrier()
    pipeline()
    pltpu.sync_copy(acc_sh.at[pl.ds(base, _ROWS_PER_TILE)],
                    outb_hbm.at[pl.ds(c * _NPAD + base, _ROWS_PER_TILE)])


_agg_sp2 = pl.kernel(
    _agg_sp2_kernel,
    out_type=(jax.ShapeDtypeStruct((2 * _NPAD, 64), jnp.float32),
              jax.ShapeDtypeStruct((2 * _NPAD, 64), jnp.float32)),
    mesh=_MESH,
    scratch_types=[
        pltpu.VMEM((_K, _G), jnp.int32),
        pltpu.VMEM((_K, _G), jnp.int32),
        pltpu.VMEM((_NBUF, _G, 64), jnp.float32),
        pltpu.VMEM_SHARED((_NPAD, 64), jnp.float32),
        pltpu.VMEM_SHARED((_NPAD, 64), jnp.float32),
        pltpu.SemaphoreType.DMA((_NBUF,)),
        pltpu.SemaphoreType.DMA((_NBUF,)),
    ],
    compiler_params=pltpu.CompilerParams(use_tc_tiling_on_sc=False),
)


def _dinv(deg_ref):
    degsum = deg_ref[0:_N, 0:1] + deg_ref[_NPAD:_NPAD + _N, 0:1] + 1.0
    return lax.rsqrt(degsum)


def _tc_scale_xw(deg_ref, x_ref, w1_ref, hsa_ref, hsb_ref):
    xw = jnp.dot(x_ref[...], w1_ref[...], preferred_element_type=jnp.float32)
    hs = xw * _dinv(deg_ref)
    zpad = jnp.zeros((_NPAD - _N, 64), jnp.float32)
    hsa_ref[0:_N, :] = hs[:, 0:64]
    hsa_ref[_N:_NPAD, :] = zpad
    hsb_ref[0:_N, :] = hs[:, 64:128]
    hsb_ref[_N:_NPAD, :] = zpad


def _tc_mid(deg_ref, pa_ref, pb_ref, hsa_ref, hsb_ref, b1_ref, w2_ref,
            out_ref):
    dinv = _dinv(deg_ref)
    agg_a = pa_ref[0:_N, :] + pa_ref[_NPAD:_NPAD + _N, :] + hsa_ref[0:_N, :]
    agg_b = pb_ref[0:_N, :] + pb_ref[_NPAD:_NPAD + _N, :] + hsb_ref[0:_N, :]
    agg = jnp.concatenate([agg_a, agg_b], axis=1)
    h = jnp.maximum(agg * dinv + b1_ref[...], 0.0)
    out_ref[0:_N, :] = jnp.dot(h, w2_ref[...],
                               preferred_element_type=jnp.float32) * dinv
    out_ref[_N:_NPAD, :] = jnp.zeros((_NPAD - _N, 64), jnp.float32)


def _tc_final(deg_ref, p_ref, h2s_ref, b2_ref, out_ref):
    dinv = _dinv(deg_ref)
    z = (p_ref[0:_N, :] + p_ref[_NPAD:_NPAD + _N, :] + h2s_ref[0:_N, :]) \
        * dinv + b2_ref[...]
    m = jnp.max(z, axis=1, keepdims=True)
    lse = jnp.log(jnp.sum(jnp.exp(z - m), axis=1, keepdims=True)) + m
    out_ref[...] = z - lse


def kernel(x, edge_index, W1, b1, W2, b2):
    src = edge_index[0].astype(jnp.int32)
    dst = edge_index[1].astype(jnp.int32)
    pad = _ETOT - _E
    src_p = jnp.concatenate(
        [src, jnp.zeros((pad,), jnp.int32)]).reshape(_NW, _K, _G)
    dst_p = jnp.concatenate(
        [dst, jnp.full((pad,), _N, jnp.int32)]).reshape(_NW, _K, _G)
    ones16 = jnp.ones((_G, 16), jnp.float32)
    z16 = jnp.zeros((_NPAD, 16), jnp.float32)
    z64 = jnp.zeros((_NPAD, 64), jnp.float32)

    degp = _deg_call(dst_p, ones16, z16)

    hsa, hsb = pl.pallas_call(
        _tc_scale_xw,
        out_shape=(jax.ShapeDtypeStruct((_NPAD, 64), jnp.float32),
                   jax.ShapeDtypeStruct((_NPAD, 64), jnp.float32)),
    )(degp, x, W1)

    p1a, p1b = _agg_sp2(hsa, hsb, src_p, dst_p, z64)

    h2s = pl.pallas_call(
        _tc_mid,
        out_shape=jax.ShapeDtypeStruct((_NPAD, 64), jnp.float32),
    )(degp, p1a, p1b, hsa, hsb, b1.reshape(1, 128), W2)

    p2 = _agg64_sp(h2s, src_p, dst_p, z64)

    out = pl.pallas_call(
        _tc_final,
        out_shape=jax.ShapeDtypeStruct((_N, 64), jnp.float32),
    )(degp, p2, h2s, b2.reshape(1, 64))
    return out
